# baseline (speedup 1.0000x reference)
import jax
import jax.numpy as jnp
from jax.experimental import pallas as pl

_DIN = 1024
_DOUT = 1024
_E = 8
_GK = 8
_K = 32
_GATE = _E * _GK
_RANK = _E * _K
_TN = 512


def _split3(a):
    hi = a.astype(jnp.bfloat16)
    r1 = a - hi.astype(jnp.float32)
    mid = r1.astype(jnp.bfloat16)
    lo = (r1 - mid.astype(jnp.float32)).astype(jnp.bfloat16)
    return hi, mid, lo


def _fused_kernel(x_ref, wpt_ref, gt_ref, mt_ref, uf_ref, eb_ref, bp_ref, o_ref):
    x = x_ref[...]
    xb = x.astype(jnp.bfloat16)
    g = jnp.dot(xb, gt_ref[...].astype(jnp.bfloat16),
                preferred_element_type=jnp.float32)
    t = jnp.dot(xb, mt_ref[...].astype(jnp.bfloat16),
                preferred_element_type=jnp.float32)

    gi = jax.lax.broadcasted_iota(jnp.int32, (_GATE, _E), 0) // _GK
    ge = jax.lax.broadcasted_iota(jnp.int32, (_GATE, _E), 1)
    gmat = (gi == ge).astype(jnp.bfloat16)
    gsq = g * g
    sh, sm, sl = _split3(gsq)
    d = lambda u: jnp.dot(u, gmat, preferred_element_type=jnp.float32)
    ss = d(sh) + d(sm) + d(sl)
    logits = jnp.sqrt(ss)

    iota = jax.lax.broadcasted_iota(jnp.int32, logits.shape, 1)
    m1 = jnp.max(logits, axis=1, keepdims=True)
    i1 = jnp.min(jnp.where(logits == m1, iota, _E), axis=1, keepdims=True)
    lm = jnp.where(iota == i1, -jnp.inf, logits)
    m2 = jnp.max(lm, axis=1, keepdims=True)
    i2 = jnp.min(jnp.where(lm == m2, iota, _E), axis=1, keepdims=True)
    p2 = jnp.exp(m2 - m1)
    denom = 1.0 + p2
    w = jnp.where(iota == i1, 1.0, jnp.where(iota == i2, p2, 0.0)) / denom

    ri = jax.lax.broadcasted_iota(jnp.int32, (_E, _RANK), 0)
    rc = jax.lax.broadcasted_iota(jnp.int32, (_E, _RANK), 1) // _K
    emat = (ri == rc).astype(jnp.float32)
    wrep = jnp.dot(w, emat, preferred_element_type=jnp.float32)
    tw = t * wrep

    out = jnp.dot(xb, wpt_ref[...].astype(jnp.bfloat16),
                  preferred_element_type=jnp.float32)
    out += jnp.dot(tw.astype(jnp.bfloat16), uf_ref[...].astype(jnp.bfloat16),
                   preferred_element_type=jnp.float32)
    out += jnp.dot(w, eb_ref[...], preferred_element_type=jnp.float32)
    out += bp_ref[...]
    o_ref[...] = out


def kernel(hidden_states, Wp, bp, gate_w, U, SVH, Eb):
    x = hidden_states.reshape(-1, _DIN)
    n = x.shape[0]
    gt = gate_w.reshape(_GATE, _DIN).T
    mt = SVH.reshape(_RANK, _DIN).T
    uf = jnp.transpose(U, (0, 2, 1)).reshape(_RANK, _DOUT)
    wpt = Wp.T
    bp2 = bp.reshape(1, _DOUT)
    out = pl.pallas_call(
        _fused_kernel,
        grid=(n // _TN,),
        in_specs=[
            pl.BlockSpec((_TN, _DIN), lambda i: (i, 0)),
            pl.BlockSpec((_DIN, _DOUT), lambda i: (0, 0)),
            pl.BlockSpec((_DIN, _GATE), lambda i: (0, 0)),
            pl.BlockSpec((_DIN, _RANK), lambda i: (0, 0)),
            pl.BlockSpec((_RANK, _DOUT), lambda i: (0, 0)),
            pl.BlockSpec((_E, _DOUT), lambda i: (0, 0)),
            pl.BlockSpec((1, _DOUT), lambda i: (0, 0)),
        ],
        out_specs=pl.BlockSpec((_TN, _DOUT), lambda i: (i, 0)),
        out_shape=jax.ShapeDtypeStruct((n, _DOUT), jnp.float32),
    )(x, wpt, gt, mt, uf, Eb, bp2)
    return out.reshape(*hidden_states.shape[:-1], _DOUT)


# speedup vs baseline: 1.0043x; 1.0043x over previous
import jax
import jax.numpy as jnp
from jax.experimental import pallas as pl

_DIN = 1024
_DOUT = 1024
_E = 8
_GK = 8
_K = 32
_GATE = _E * _GK
_RANK = _E * _K
_TN = 512


def _split3(a):
    hi = a.astype(jnp.bfloat16)
    r1 = a - hi.astype(jnp.float32)
    mid = r1.astype(jnp.bfloat16)
    lo = (r1 - mid.astype(jnp.float32)).astype(jnp.bfloat16)
    return hi, mid, lo


def _fused_kernel(x_ref, wpt_ref, gt_ref, mt_ref, uf_ref, eb_ref, bp_ref, o_ref):
    x = x_ref[...]
    xb = x.astype(jnp.bfloat16)
    g = jnp.dot(xb, gt_ref[...],
                preferred_element_type=jnp.float32)
    t = jnp.dot(xb, mt_ref[...], preferred_element_type=jnp.float32)

    gi = jax.lax.broadcasted_iota(jnp.int32, (_GATE, _E), 0) // _GK
    ge = jax.lax.broadcasted_iota(jnp.int32, (_GATE, _E), 1)
    gmat = (gi == ge).astype(jnp.bfloat16)
    gsq = g * g
    sh, sm, sl = _split3(gsq)
    d = lambda u: jnp.dot(u, gmat, preferred_element_type=jnp.float32)
    ss = d(sh) + d(sm) + d(sl)
    logits = jnp.sqrt(ss)

    iota = jax.lax.broadcasted_iota(jnp.int32, logits.shape, 1)
    m1 = jnp.max(logits, axis=1, keepdims=True)
    i1 = jnp.min(jnp.where(logits == m1, iota, _E), axis=1, keepdims=True)
    lm = jnp.where(iota == i1, -jnp.inf, logits)
    m2 = jnp.max(lm, axis=1, keepdims=True)
    i2 = jnp.min(jnp.where(lm == m2, iota, _E), axis=1, keepdims=True)
    p2 = jnp.exp(m2 - m1)
    denom = 1.0 + p2
    w = jnp.where(iota == i1, 1.0, jnp.where(iota == i2, p2, 0.0)) / denom

    ri = jax.lax.broadcasted_iota(jnp.int32, (_E, _RANK), 0)
    rc = jax.lax.broadcasted_iota(jnp.int32, (_E, _RANK), 1) // _K
    emat = (ri == rc).astype(jnp.float32)
    wrep = jnp.dot(w, emat, preferred_element_type=jnp.float32)
    tw = t * wrep

    out = jnp.dot(xb, wpt_ref[...], preferred_element_type=jnp.float32)
    out += jnp.dot(tw.astype(jnp.bfloat16), uf_ref[...],
                   preferred_element_type=jnp.float32)
    out += jnp.dot(w, eb_ref[...], preferred_element_type=jnp.float32)
    out += bp_ref[...]
    o_ref[...] = out


def kernel(hidden_states, Wp, bp, gate_w, U, SVH, Eb):
    x = hidden_states.reshape(-1, _DIN)
    n = x.shape[0]
    gt = gate_w.reshape(_GATE, _DIN).T.astype(jnp.bfloat16)
    mt = SVH.reshape(_RANK, _DIN).T.astype(jnp.bfloat16)
    uf = jnp.transpose(U, (0, 2, 1)).reshape(_RANK, _DOUT).astype(jnp.bfloat16)
    wpt = Wp.T.astype(jnp.bfloat16)
    bp2 = bp.reshape(1, _DOUT)
    out = pl.pallas_call(
        _fused_kernel,
        grid=(n // _TN,),
        in_specs=[
            pl.BlockSpec((_TN, _DIN), lambda i: (i, 0)),
            pl.BlockSpec((_DIN, _DOUT), lambda i: (0, 0)),
            pl.BlockSpec((_DIN, _GATE), lambda i: (0, 0)),
            pl.BlockSpec((_DIN, _RANK), lambda i: (0, 0)),
            pl.BlockSpec((_RANK, _DOUT), lambda i: (0, 0)),
            pl.BlockSpec((_E, _DOUT), lambda i: (0, 0)),
            pl.BlockSpec((1, _DOUT), lambda i: (0, 0)),
        ],
        out_specs=pl.BlockSpec((_TN, _DOUT), lambda i: (i, 0)),
        out_shape=jax.ShapeDtypeStruct((n, _DOUT), jnp.float32),
    )(x, wpt, gt, mt, uf, Eb, bp2)
    return out.reshape(*hidden_states.shape[:-1], _DOUT)


# speedup vs baseline: 1.0149x; 1.0106x over previous
import jax
import jax.numpy as jnp
from jax.experimental import pallas as pl

_DIN = 1024
_DOUT = 1024
_E = 8
_GK = 8
_K = 32
_GATE = _E * _GK
_RANK = _E * _K
_TN = 512


def _split3(a):
    hi = a.astype(jnp.bfloat16)
    r1 = a - hi.astype(jnp.float32)
    mid = r1.astype(jnp.bfloat16)
    lo = (r1 - mid.astype(jnp.float32)).astype(jnp.bfloat16)
    return hi, mid, lo


def _fused_kernel(x_ref, wpt_ref, gt_ref, mt_ref, uf_ref, eb_ref, bp_ref, o_ref):
    x = x_ref[...]
    xb = x.astype(jnp.bfloat16)
    _dnt = (((1,), (1,)), ((), ()))
    g = jax.lax.dot_general(xb, gt_ref[...], _dnt,
                            preferred_element_type=jnp.float32)
    t = jax.lax.dot_general(xb, mt_ref[...], _dnt,
                            preferred_element_type=jnp.float32)

    gi = jax.lax.broadcasted_iota(jnp.int32, (_GATE, _E), 0) // _GK
    ge = jax.lax.broadcasted_iota(jnp.int32, (_GATE, _E), 1)
    gmat = (gi == ge).astype(jnp.bfloat16)
    gsq = g * g
    sh, sm, sl = _split3(gsq)
    d = lambda u: jnp.dot(u, gmat, preferred_element_type=jnp.float32)
    ss = d(sh) + d(sm) + d(sl)
    logits = jnp.sqrt(ss)

    iota = jax.lax.broadcasted_iota(jnp.int32, logits.shape, 1)
    m1 = jnp.max(logits, axis=1, keepdims=True)
    i1 = jnp.min(jnp.where(logits == m1, iota, _E), axis=1, keepdims=True)
    lm = jnp.where(iota == i1, -jnp.inf, logits)
    m2 = jnp.max(lm, axis=1, keepdims=True)
    i2 = jnp.min(jnp.where(lm == m2, iota, _E), axis=1, keepdims=True)
    p2 = jnp.exp(m2 - m1)
    denom = 1.0 + p2
    w = jnp.where(iota == i1, 1.0, jnp.where(iota == i2, p2, 0.0)) / denom

    ri = jax.lax.broadcasted_iota(jnp.int32, (_E, _RANK), 0)
    rc = jax.lax.broadcasted_iota(jnp.int32, (_E, _RANK), 1) // _K
    emat = (ri == rc).astype(jnp.float32)
    wrep = jnp.dot(w, emat, preferred_element_type=jnp.float32)
    tw = t * wrep

    out = jax.lax.dot_general(xb, wpt_ref[...], _dnt,
                              preferred_element_type=jnp.float32)
    out += jnp.dot(tw.astype(jnp.bfloat16), uf_ref[...],
                   preferred_element_type=jnp.float32)
    out += jnp.dot(w, eb_ref[...], preferred_element_type=jnp.float32)
    out += bp_ref[...]
    o_ref[...] = out


def kernel(hidden_states, Wp, bp, gate_w, U, SVH, Eb):
    x = hidden_states.reshape(-1, _DIN)
    n = x.shape[0]
    gt = gate_w.reshape(_GATE, _DIN).astype(jnp.bfloat16)
    mt = SVH.reshape(_RANK, _DIN).astype(jnp.bfloat16)
    uf = jnp.transpose(U, (0, 2, 1)).reshape(_RANK, _DOUT).astype(jnp.bfloat16)
    wpt = Wp.astype(jnp.bfloat16)
    bp2 = bp.reshape(1, _DOUT)
    out = pl.pallas_call(
        _fused_kernel,
        grid=(n // _TN,),
        in_specs=[
            pl.BlockSpec((_TN, _DIN), lambda i: (i, 0)),
            pl.BlockSpec((_DOUT, _DIN), lambda i: (0, 0)),
            pl.BlockSpec((_GATE, _DIN), lambda i: (0, 0)),
            pl.BlockSpec((_RANK, _DIN), lambda i: (0, 0)),
            pl.BlockSpec((_RANK, _DOUT), lambda i: (0, 0)),
            pl.BlockSpec((_E, _DOUT), lambda i: (0, 0)),
            pl.BlockSpec((1, _DOUT), lambda i: (0, 0)),
        ],
        out_specs=pl.BlockSpec((_TN, _DOUT), lambda i: (i, 0)),
        out_shape=jax.ShapeDtypeStruct((n, _DOUT), jnp.float32),
    )(x, wpt, gt, mt, uf, Eb, bp2)
    return out.reshape(*hidden_states.shape[:-1], _DOUT)


# speedup vs baseline: 1.0743x; 1.0586x over previous
import jax
import jax.numpy as jnp
from jax.experimental import pallas as pl

_DIN = 1024
_DOUT = 1024
_E = 8
_GK = 8
_K = 32
_GATE = _E * _GK
_RANK = _E * _K
_TN = 1024


def _split3(a):
    hi = a.astype(jnp.bfloat16)
    r1 = a - hi.astype(jnp.float32)
    mid = r1.astype(jnp.bfloat16)
    lo = (r1 - mid.astype(jnp.float32)).astype(jnp.bfloat16)
    return hi, mid, lo


def _fused_kernel(x_ref, wpt_ref, gt_ref, mt_ref, uf_ref, eb_ref, bp_ref, o_ref):
    x = x_ref[...]
    xb = x.astype(jnp.bfloat16)
    _dnt = (((1,), (1,)), ((), ()))
    g = jax.lax.dot_general(xb, gt_ref[...], _dnt,
                            preferred_element_type=jnp.float32)
    t = jax.lax.dot_general(xb, mt_ref[...], _dnt,
                            preferred_element_type=jnp.float32)

    gi = jax.lax.broadcasted_iota(jnp.int32, (_GATE, _E), 0) // _GK
    ge = jax.lax.broadcasted_iota(jnp.int32, (_GATE, _E), 1)
    gmat = (gi == ge).astype(jnp.bfloat16)
    gsq = g * g
    sh, sm, sl = _split3(gsq)
    d = lambda u: jnp.dot(u, gmat, preferred_element_type=jnp.float32)
    ss = d(sh) + d(sm) + d(sl)
    logits = jnp.sqrt(ss)

    iota = jax.lax.broadcasted_iota(jnp.int32, logits.shape, 1)
    m1 = jnp.max(logits, axis=1, keepdims=True)
    i1 = jnp.min(jnp.where(logits == m1, iota, _E), axis=1, keepdims=True)
    lm = jnp.where(iota == i1, -jnp.inf, logits)
    m2 = jnp.max(lm, axis=1, keepdims=True)
    i2 = jnp.min(jnp.where(lm == m2, iota, _E), axis=1, keepdims=True)
    p2 = jnp.exp(m2 - m1)
    denom = 1.0 + p2
    w = jnp.where(iota == i1, 1.0, jnp.where(iota == i2, p2, 0.0)) / denom

    ri = jax.lax.broadcasted_iota(jnp.int32, (_E, _RANK), 0)
    rc = jax.lax.broadcasted_iota(jnp.int32, (_E, _RANK), 1) // _K
    emat = (ri == rc).astype(jnp.float32)
    wrep = jnp.dot(w, emat, preferred_element_type=jnp.float32)
    tw = t * wrep

    out = jax.lax.dot_general(xb, wpt_ref[...], _dnt,
                              preferred_element_type=jnp.float32)
    out += jnp.dot(tw.astype(jnp.bfloat16), uf_ref[...],
                   preferred_element_type=jnp.float32)
    out += jnp.dot(w, eb_ref[...], preferred_element_type=jnp.float32)
    out += bp_ref[...]
    o_ref[...] = out


def kernel(hidden_states, Wp, bp, gate_w, U, SVH, Eb):
    x = hidden_states.reshape(-1, _DIN)
    n = x.shape[0]
    gt = gate_w.reshape(_GATE, _DIN).astype(jnp.bfloat16)
    mt = SVH.reshape(_RANK, _DIN).astype(jnp.bfloat16)
    uf = jnp.transpose(U, (0, 2, 1)).reshape(_RANK, _DOUT).astype(jnp.bfloat16)
    wpt = Wp.astype(jnp.bfloat16)
    bp2 = bp.reshape(1, _DOUT)
    out = pl.pallas_call(
        _fused_kernel,
        grid=(n // _TN,),
        in_specs=[
            pl.BlockSpec((_TN, _DIN), lambda i: (i, 0)),
            pl.BlockSpec((_DOUT, _DIN), lambda i: (0, 0)),
            pl.BlockSpec((_GATE, _DIN), lambda i: (0, 0)),
            pl.BlockSpec((_RANK, _DIN), lambda i: (0, 0)),
            pl.BlockSpec((_RANK, _DOUT), lambda i: (0, 0)),
            pl.BlockSpec((_E, _DOUT), lambda i: (0, 0)),
            pl.BlockSpec((1, _DOUT), lambda i: (0, 0)),
        ],
        out_specs=pl.BlockSpec((_TN, _DOUT), lambda i: (i, 0)),
        out_shape=jax.ShapeDtypeStruct((n, _DOUT), jnp.float32),
    )(x, wpt, gt, mt, uf, Eb, bp2)
    return out.reshape(*hidden_states.shape[:-1], _DOUT)
